# Initial kernel scaffold; baseline (speedup 1.0000x reference)
import jax
import jax.numpy as jnp
from jax.experimental import pallas as pl


def kernel(emb_user, emb_item, eigs, lambda0s, path_embs, indices, path_type):
    raise NotImplementedError("write your pallas kernel here")


# speedup vs baseline: 2.8927x; 2.8927x over previous
import functools

import jax
import jax.numpy as jnp
from jax import lax
from jax.experimental import pallas as pl
from jax.experimental.pallas import tpu as pltpu
from jax.experimental.pallas import tpu_sc as plsc

N_USERS = 5000
N_ITEMS = 5000
N = N_USERS + N_ITEMS
E = 160000
D = 256
EIGS_DIM = 16
N_LAYERS = 2
N_PATHS = 6
SQRT_DIM = 0.0625
EXP_OFF = 16.0

NC = 2
NS = 16
L = 16
NW = NC * NS


_LN_ROWS = 1000


def _ln_body(x_ref, o_ref):
    x = x_ref[...]
    mu = jnp.mean(x, axis=-1, keepdims=True)
    var = jnp.mean((x - mu) ** 2, axis=-1, keepdims=True)
    o_ref[...] = (x - mu) * lax.rsqrt(var + 1e-5)


def _layernorm(x):
    return pl.pallas_call(
        _ln_body,
        out_shape=jax.ShapeDtypeStruct((N, D), jnp.float32),
        grid=(N // _LN_ROWS,),
        in_specs=[pl.BlockSpec((_LN_ROWS, D), lambda i: (i, 0))],
        out_specs=pl.BlockSpec((_LN_ROWS, D), lambda i: (i, 0)),
    )(x)


def _combine_body(p0_ref, p1_ref, o_ref):
    s0 = jnp.sum(p0_ref[...], axis=0)
    s1 = jnp.sum(p1_ref[...], axis=0)
    o_ref[0, :] = 1.0 / (s0 + 1e-12)
    o_ref[1, :] = 1.0 / (s1 + 1e-12)


def _combine(p0, p1):
    return pl.pallas_call(
        _combine_body,
        out_shape=jax.ShapeDtypeStruct((2, N), jnp.float32),
    )(p0, p1)


def _mean_body(a_ref, b_ref, c_ref, o_ref):
    o_ref[...] = (a_ref[...] + b_ref[...] + c_ref[...]) * (1.0 / 3.0)


def _mean3(a, b, c):
    return pl.pallas_call(
        _mean_body,
        out_shape=jax.ShapeDtypeStruct((N, D), jnp.float32),
        grid=(N // _LN_ROWS,),
        in_specs=[pl.BlockSpec((_LN_ROWS, D), lambda i: (i, 0))] * 3,
        out_specs=pl.BlockSpec((_LN_ROWS, D), lambda i: (i, 0)),
    )(a, b, c)


EC2 = E // NW
B2 = 40
NB2 = EC2 // B2
B2P = 48

_sc_mesh = plsc.VectorSubcoreMesh(
    core_axis_name="c", subcore_axis_name="s", num_cores=NC, num_subcores=NS)


def _edge_body(y_hbm, eigs0_hbm, eigs1_hbm, i0_hbm, i1_hbm, pt_hbm, pemb_hbm,
               e0_hbm, e1_hbm, p0_hbm, p1_hbm,
               i0_v, i1_v, pt_v, rows0, rows1, er0, er1,
               pemb_v, e0buf, e1buf, sums0, sums1, sem0, sem1):
    cid = lax.axis_index("c")
    sid = lax.axis_index("s")
    wid = sid * NC + cid

    pltpu.sync_copy(i0_hbm.at[wid], i0_v)
    pltpu.sync_copy(i1_hbm.at[wid], i1_v)
    pltpu.sync_copy(pt_hbm.at[wid], pt_v)
    pltpu.sync_copy(pemb_hbm, pemb_v)

    zero = jnp.zeros((L,), jnp.float32)

    def _zero(i, _):
        sums0[pl.ds(i * L, L)] = zero
        sums1[pl.ds(i * L, L)] = zero
        return 0

    lax.fori_loop(0, N // L, _zero, 0)

    lane = jax.lax.iota(jnp.int32, L)
    _GROUPS2 = [(0, None), (L, None), (B2 - L, lane >= (2 * L - (B2 - L)))]

    def _batch(j, _):
        g0 = pltpu.async_copy(y_hbm.at[i0_v.at[j]], rows0, sem0)
        g1 = pltpu.async_copy(y_hbm.at[i1_v.at[j]], rows1, sem1)
        g0.wait()
        g1.wait()
        g0 = pltpu.async_copy(eigs0_hbm.at[i0_v.at[j]], er0, sem0)
        g1 = pltpu.async_copy(eigs1_hbm.at[i1_v.at[j]], er1, sem1)
        g0.wait()
        g1.wait()

        for start, mk in _GROUPS2:
            ev = lane + start

            def _dot(k, acc):
                kv = jnp.full((L,), 0, jnp.int32) + k
                g0v = plsc.load_gather(rows0, [ev, kv])
                g1v = plsc.load_gather(rows1, [ev, kv])
                return acc + g0v * g1v

            x = lax.fori_loop(0, D, _dot, jnp.zeros((L,), jnp.float32),
                              unroll=8)

            def _edot(k, acc):
                kv = jnp.full((L,), 0, jnp.int32) + k
                g0v = plsc.load_gather(er0, [ev, kv])
                g1v = plsc.load_gather(er1, [ev, kv])
                return acc + g0v * g1v

            ye = lax.fori_loop(0, EIGS_DIM, _edot,
                               jnp.zeros((L,), jnp.float32), unroll=8)

            s0v = x * SQRT_DIM + ye
            e0v = jnp.exp(s0v - EXP_OFF)
            sl_chunk = pl.ds(start, L)
            ptv = pt_v[j, sl_chunk]
            zv = plsc.load_gather(pemb_v, [ptv])
            e1v = jnp.exp(zv)
            if mk is None:
                e0buf[sl_chunk] = e0v
                e1buf[sl_chunk] = e1v
            else:
                e0buf[sl_chunk] = jnp.where(mk, e0v, e0buf[sl_chunk])
                e1buf[sl_chunk] = jnp.where(mk, e1v, e1buf[sl_chunk])
            i0v = i0_v[j, sl_chunk]
            plsc.addupdate_scatter(sums0, [i0v], e0v, mask=mk)
            plsc.addupdate_scatter(sums1, [i0v], e1v, mask=mk)

        pltpu.sync_copy(e0buf, e0_hbm.at[wid, j])
        pltpu.sync_copy(e1buf, e1_hbm.at[wid, j])
        return 0

    lax.fori_loop(0, NB2, _batch, 0)

    pltpu.sync_copy(sums0, p0_hbm.at[wid])
    pltpu.sync_copy(sums1, p1_hbm.at[wid])


_edge_kernel = functools.partial(
    pl.kernel,
    out_type=(
        jax.ShapeDtypeStruct((NW, NB2, B2), jnp.float32),
        jax.ShapeDtypeStruct((NW, NB2, B2), jnp.float32),
        jax.ShapeDtypeStruct((NW, N), jnp.float32),
        jax.ShapeDtypeStruct((NW, N), jnp.float32),
    ),
    mesh=_sc_mesh,
    scratch_types=[
        pltpu.VMEM((NB2, B2), jnp.int32),
        pltpu.VMEM((NB2, B2), jnp.int32),
        pltpu.VMEM((NB2, B2), jnp.int32),
        pltpu.VMEM((B2, D), jnp.float32),
        pltpu.VMEM((B2, D), jnp.float32),
        pltpu.VMEM((B2, EIGS_DIM), jnp.float32),
        pltpu.VMEM((B2, EIGS_DIM), jnp.float32),
        pltpu.VMEM((L,), jnp.float32),
        pltpu.VMEM((B2,), jnp.float32),
        pltpu.VMEM((B2,), jnp.float32),
        pltpu.VMEM((N,), jnp.float32),
        pltpu.VMEM((N,), jnp.float32),
        pltpu.SemaphoreType.DMA,
        pltpu.SemaphoreType.DMA,
    ],
    compiler_params=pltpu.CompilerParams(use_tc_tiling_on_sc=False, needs_layout_passes=False),
)(_edge_body)


EC4 = E // NS
B4 = 80
NB4 = EC4 // B4
NQ = 4
DH = D // NQ
RPS = N // NS
RCH = 125


def _spmm_body(cb, y4_hbm, i0_hbm, i1_hbm, e0_hbm, e1_hbm, inv_hbm,
               out_hbm,
               i0_v, i1_v, e0_v, e1_v, inv0_v, inv1_v,
               ridx, rows, zbuf, acc_sh, sem0):
    cid = lax.axis_index("c")
    sid = lax.axis_index("s")
    qid = cb * NC + cid

    zero = jnp.zeros((L,), jnp.float32)

    def _zero(i, _):
        zbuf[i // (DH // L), pl.ds((i % (DH // L)) * L, L)] = zero
        return 0

    lax.fori_loop(0, (RCH * DH) // L, _zero, 0)
    for r in range(RPS // RCH):
        pltpu.sync_copy(zbuf, acc_sh.at[pl.ds(sid * RPS + r * RCH, RCH)])
    plsc.subcore_barrier()

    pltpu.sync_copy(i0_hbm.at[sid], i0_v)
    pltpu.sync_copy(i1_hbm.at[sid], i1_v)
    pltpu.sync_copy(e0_hbm.at[sid], e0_v)
    pltpu.sync_copy(e1_hbm.at[sid], e1_v)
    pltpu.sync_copy(inv_hbm.at[0], inv0_v)
    pltpu.sync_copy(inv_hbm.at[1], inv1_v)

    def _batch(j, _):
        for t in range(B4 // L):
            i1v = i1_v[j, pl.ds(t * L, L)]
            ridx[pl.ds(t * L, L)] = i1v * NQ + qid
        pltpu.async_copy(y4_hbm.at[ridx], rows, sem0).wait()

        for t in range(B4 // L):
            sl_chunk = pl.ds(t * L, L)
            i0v = i0_v[j, sl_chunk]
            a0 = plsc.load_gather(inv0_v, [i0v])
            a1 = plsc.load_gather(inv1_v, [i0v])
            sv = 0.5 * (e0_v[j, sl_chunk] * a0 + e1_v[j, sl_chunk] * a1)
            for e in range(L):
                se = sv[e]
                re = t * L + e
                for k in range(DH // L):
                    rows[re, pl.ds(k * L, L)] = rows[re, pl.ds(k * L, L)] * se

        pltpu.sync_copy(rows, acc_sh.at[i0_v.at[j]], add=True)
        return 0

    lax.fori_loop(0, NB4, _batch, 0)
    plsc.subcore_barrier()

    for r in range(RPS // RCH):
        row0 = sid * RPS + r * RCH
        pltpu.sync_copy(
            acc_sh.at[pl.ds(row0, RCH)],
            out_hbm.at[(pl.ds(row0, RCH), pl.ds(cid * DH, DH))])


def _make_spmm(cb):
    return functools.partial(
        pl.kernel,
        out_type=jax.ShapeDtypeStruct((N, D // 2), jnp.float32),
        mesh=_sc_mesh,
        scratch_types=[
            pltpu.VMEM((NB4, B4), jnp.int32),
            pltpu.VMEM((NB4, B4), jnp.int32),
            pltpu.VMEM((NB4, B4), jnp.float32),
            pltpu.VMEM((NB4, B4), jnp.float32),
            pltpu.VMEM((N,), jnp.float32),
            pltpu.VMEM((N,), jnp.float32),
            pltpu.VMEM((B4,), jnp.int32),
            pltpu.VMEM((B4, DH), jnp.float32),
            pltpu.VMEM((RCH, DH), jnp.float32),
            pltpu.VMEM_SHARED((N, DH), jnp.float32),
            pltpu.SemaphoreType.DMA,
        ],
        compiler_params=pltpu.CompilerParams(
            use_tc_tiling_on_sc=False, needs_layout_passes=False),
    )(functools.partial(_spmm_body, cb))


_spmm_kernels = [_make_spmm(0), _make_spmm(1)]


def kernel(emb_user, emb_item, eigs, lambda0s, path_embs, indices, path_type):
    all_emb = jnp.concatenate([emb_user, emb_item], axis=0)
    embs = [all_emb]
    for l in range(N_LAYERS):
        y = _layernorm(all_emb)
        i0 = indices[l, 0]
        i1 = indices[l, 1]
        pt = path_type[l]
        i0_2 = i0.reshape(NW, NB2, B2)
        i1_2 = i1.reshape(NW, NB2, B2)
        pt_2 = pt.reshape(NW, NB2, B2)
        eigs0 = eigs * jnp.exp(lambda0s[l])
        pemb = jnp.zeros((L,), jnp.float32).at[:N_PATHS].set(
            path_embs[l].reshape(-1))
        e0, e1, p0, p1 = _edge_kernel(y, eigs0, eigs, i0_2, i1_2, pt_2, pemb)
        inv = _combine(p0, p1)
        y4 = y.reshape(N * NQ, DH)
        i0_4 = i0.reshape(NS, NB4, B4)
        i1_4 = i1.reshape(NS, NB4, B4)
        e0_4 = e0.reshape(NS, NB4, B4)
        e1_4 = e1.reshape(NS, NB4, B4)
        outs = [k(y4, i0_4, i1_4, e0_4, e1_4, inv) for k in _spmm_kernels]
        all_emb = jnp.concatenate(outs, axis=1)
        embs.append(all_emb)
    return _mean3(embs[0], embs[1], embs[2])


# speedup vs baseline: 5.6252x; 1.9446x over previous
import functools

import jax
import jax.numpy as jnp
from jax import lax
from jax.experimental import pallas as pl
from jax.experimental.pallas import tpu as pltpu
from jax.experimental.pallas import tpu_sc as plsc

N_USERS = 5000
N_ITEMS = 5000
N = N_USERS + N_ITEMS
E = 160000
D = 256
EIGS_DIM = 16
N_LAYERS = 2
N_PATHS = 6
SQRT_DIM = 0.0625
EXP_OFF = 16.0

NC = 2
NS = 16
L = 16
NW = NC * NS


_LN_ROWS = 1000


DA = D + EIGS_DIM


def _ln_body(x_ref, es_ref, e_ref, a0_ref, a1_ref, y_ref):
    x = x_ref[...]
    mu = jnp.mean(x, axis=-1, keepdims=True)
    var = jnp.mean((x - mu) ** 2, axis=-1, keepdims=True)
    y = (x - mu) * lax.rsqrt(var + 1e-5)
    y_ref[...] = y
    ys = y * 0.25
    a0_ref[:, :D] = ys
    a0_ref[:, D:] = es_ref[...]
    a1_ref[:, :D] = ys
    a1_ref[:, D:] = e_ref[...]


def _ln_aug(x, eigs_s, eigs):
    return pl.pallas_call(
        _ln_body,
        out_shape=(
            jax.ShapeDtypeStruct((N, DA), jnp.float32),
            jax.ShapeDtypeStruct((N, DA), jnp.float32),
            jax.ShapeDtypeStruct((N, D), jnp.float32),
        ),
        grid=(N // _LN_ROWS,),
        in_specs=[
            pl.BlockSpec((_LN_ROWS, D), lambda i: (i, 0)),
            pl.BlockSpec((_LN_ROWS, EIGS_DIM), lambda i: (i, 0)),
            pl.BlockSpec((_LN_ROWS, EIGS_DIM), lambda i: (i, 0)),
        ],
        out_specs=(
            pl.BlockSpec((_LN_ROWS, DA), lambda i: (i, 0)),
            pl.BlockSpec((_LN_ROWS, DA), lambda i: (i, 0)),
            pl.BlockSpec((_LN_ROWS, D), lambda i: (i, 0)),
        ),
    )(x, eigs_s, eigs)


def _combine_body(p0_ref, p1_ref, o_ref):
    s0 = jnp.sum(p0_ref[...], axis=0)
    s1 = jnp.sum(p1_ref[...], axis=0)
    o_ref[0, :] = 1.0 / (s0 + 1e-12)
    o_ref[1, :] = 1.0 / (s1 + 1e-12)


def _combine(p0, p1):
    return pl.pallas_call(
        _combine_body,
        out_shape=jax.ShapeDtypeStruct((2, N), jnp.float32),
    )(p0, p1)


def _mean_body(a_ref, b_ref, c_ref, o_ref):
    o_ref[...] = (a_ref[...] + b_ref[...] + c_ref[...]) * (1.0 / 3.0)


def _mean3(a, b, c):
    return pl.pallas_call(
        _mean_body,
        out_shape=jax.ShapeDtypeStruct((N, D), jnp.float32),
        grid=(N // _LN_ROWS,),
        in_specs=[pl.BlockSpec((_LN_ROWS, D), lambda i: (i, 0))] * 3,
        out_specs=pl.BlockSpec((_LN_ROWS, D), lambda i: (i, 0)),
    )(a, b, c)


EC2 = E // NW
B2 = 40
NB2 = EC2 // B2
B2P = 48

_sc_mesh = plsc.VectorSubcoreMesh(
    core_axis_name="c", subcore_axis_name="s", num_cores=NC, num_subcores=NS)


def _edge_body(aug0_hbm, aug1_hbm, i0_hbm, i1_hbm, pt_hbm, pemb_hbm,
               e0_hbm, e1_hbm, p0_hbm, p1_hbm,
               i0_v, i1_v, pt_v, rowsA0, rowsA1, rowsB0, rowsB1,
               pemb_v, e0buf, e1buf, sums0, sums1, sA0, sA1, sB0, sB1):
    cid = lax.axis_index("c")
    sid = lax.axis_index("s")
    wid = sid * NC + cid

    pltpu.sync_copy(i0_hbm.at[wid], i0_v)
    pltpu.sync_copy(i1_hbm.at[wid], i1_v)
    pltpu.sync_copy(pt_hbm.at[wid], pt_v)
    pltpu.sync_copy(pemb_hbm, pemb_v)

    zero = jnp.zeros((L,), jnp.float32)

    def _zero(i, _):
        sums0[pl.ds(i * L, L)] = zero
        sums1[pl.ds(i * L, L)] = zero
        return 0

    lax.fori_loop(0, N // L, _zero, 0)

    lane = jax.lax.iota(jnp.int32, L)
    groups = [(0, None), (L, None), (B2 - L, lane >= (3 * L - B2))]

    def _issue(j, r0, r1, s0, s1):
        pltpu.async_copy(aug0_hbm.at[i0_v.at[j]], r0, s0)
        pltpu.async_copy(aug1_hbm.at[i1_v.at[j]], r1, s1)

    def _wait(j, r0, r1, s0, s1):
        pltpu.make_async_copy(aug0_hbm.at[i0_v.at[j]], r0, s0).wait()
        pltpu.make_async_copy(aug1_hbm.at[i1_v.at[j]], r1, s1).wait()

    def _compute(j, r0, r1):
        for start, mk in groups:
            ev = lane + start

            def _dot(k, acc):
                kv = jnp.full((L,), 0, jnp.int32) + k
                g0v = plsc.load_gather(r0, [ev, kv])
                g1v = plsc.load_gather(r1, [ev, kv])
                return acc + g0v * g1v

            s0v = lax.fori_loop(0, DA, _dot, jnp.zeros((L,), jnp.float32),
                                unroll=8)
            e0v = jnp.exp(s0v - EXP_OFF)
            sl_chunk = pl.ds(start, L)
            ptv = pt_v[j, sl_chunk]
            zv = plsc.load_gather(pemb_v, [ptv])
            e1v = jnp.exp(zv)
            if mk is None:
                e0buf[sl_chunk] = e0v
                e1buf[sl_chunk] = e1v
            else:
                e0buf[sl_chunk] = jnp.where(mk, e0v, e0buf[sl_chunk])
                e1buf[sl_chunk] = jnp.where(mk, e1v, e1buf[sl_chunk])
            i0v = i0_v[j, sl_chunk]
            plsc.addupdate_scatter(sums0, [i0v], e0v, mask=mk)
            plsc.addupdate_scatter(sums1, [i0v], e1v, mask=mk)

        pltpu.sync_copy(e0buf, e0_hbm.at[wid, j])
        pltpu.sync_copy(e1buf, e1_hbm.at[wid, j])

    _issue(0, rowsA0, rowsA1, sA0, sA1)

    def _pair(p, _):
        j = 2 * p

        @pl.when(j + 1 < NB2)
        def _():
            _issue(j + 1, rowsB0, rowsB1, sB0, sB1)

        _wait(j, rowsA0, rowsA1, sA0, sA1)
        _compute(j, rowsA0, rowsA1)

        @pl.when(j + 2 < NB2)
        def _():
            _issue(j + 2, rowsA0, rowsA1, sA0, sA1)

        @pl.when(j + 1 < NB2)
        def _():
            _wait(j + 1, rowsB0, rowsB1, sB0, sB1)
            _compute(j + 1, rowsB0, rowsB1)

        return 0

    lax.fori_loop(0, (NB2 + 1) // 2, _pair, 0)

    pltpu.sync_copy(sums0, p0_hbm.at[wid])
    pltpu.sync_copy(sums1, p1_hbm.at[wid])


_edge_kernel = functools.partial(
    pl.kernel,
    out_type=(
        jax.ShapeDtypeStruct((NW, NB2, B2), jnp.float32),
        jax.ShapeDtypeStruct((NW, NB2, B2), jnp.float32),
        jax.ShapeDtypeStruct((NW, N), jnp.float32),
        jax.ShapeDtypeStruct((NW, N), jnp.float32),
    ),
    mesh=_sc_mesh,
    scratch_types=[
        pltpu.VMEM((NB2, B2), jnp.int32),
        pltpu.VMEM((NB2, B2), jnp.int32),
        pltpu.VMEM((NB2, B2), jnp.int32),
        pltpu.VMEM((B2, DA), jnp.float32),
        pltpu.VMEM((B2, DA), jnp.float32),
        pltpu.VMEM((B2, DA), jnp.float32),
        pltpu.VMEM((B2, DA), jnp.float32),
        pltpu.VMEM((L,), jnp.float32),
        pltpu.VMEM((B2,), jnp.float32),
        pltpu.VMEM((B2,), jnp.float32),
        pltpu.VMEM((N,), jnp.float32),
        pltpu.VMEM((N,), jnp.float32),
        pltpu.SemaphoreType.DMA,
        pltpu.SemaphoreType.DMA,
        pltpu.SemaphoreType.DMA,
        pltpu.SemaphoreType.DMA,
    ],
    compiler_params=pltpu.CompilerParams(use_tc_tiling_on_sc=False, needs_layout_passes=False),
)(_edge_body)


EC4 = E // NS
B4 = 80
NB4 = EC4 // B4
NQ = 4
DH = D // NQ
RPS = N // NS
RCH = 125


def _spmm_body(cb, y4_hbm, i0_hbm, i1_hbm, e0_hbm, e1_hbm, inv_hbm,
               out_hbm,
               i0_v, i1_v, e0_v, e1_v, inv0_v, inv1_v,
               ridx, rows, zbuf, acc_sh, sem0):
    cid = lax.axis_index("c")
    sid = lax.axis_index("s")
    qid = cb * NC + cid

    zero = jnp.zeros((L,), jnp.float32)

    def _zero(i, _):
        zbuf[i // (DH // L), pl.ds((i % (DH // L)) * L, L)] = zero
        return 0

    lax.fori_loop(0, (RCH * DH) // L, _zero, 0)
    for r in range(RPS // RCH):
        pltpu.sync_copy(zbuf, acc_sh.at[pl.ds(sid * RPS + r * RCH, RCH)])
    plsc.subcore_barrier()

    pltpu.sync_copy(i0_hbm.at[sid], i0_v)
    pltpu.sync_copy(i1_hbm.at[sid], i1_v)
    pltpu.sync_copy(e0_hbm.at[sid], e0_v)
    pltpu.sync_copy(e1_hbm.at[sid], e1_v)
    pltpu.sync_copy(inv_hbm.at[0], inv0_v)
    pltpu.sync_copy(inv_hbm.at[1], inv1_v)

    def _batch(j, _):
        for t in range(B4 // L):
            i1v = i1_v[j, pl.ds(t * L, L)]
            ridx[pl.ds(t * L, L)] = i1v * NQ + qid
        pltpu.async_copy(y4_hbm.at[ridx], rows, sem0).wait()

        for t in range(B4 // L):
            sl_chunk = pl.ds(t * L, L)
            i0v = i0_v[j, sl_chunk]
            a0 = plsc.load_gather(inv0_v, [i0v])
            a1 = plsc.load_gather(inv1_v, [i0v])
            sv = 0.5 * (e0_v[j, sl_chunk] * a0 + e1_v[j, sl_chunk] * a1)
            for e in range(L):
                se = sv[e]
                re = t * L + e
                for k in range(DH // L):
                    rows[re, pl.ds(k * L, L)] = rows[re, pl.ds(k * L, L)] * se

        pltpu.sync_copy(rows, acc_sh.at[i0_v.at[j]], add=True)
        return 0

    lax.fori_loop(0, NB4, _batch, 0)
    plsc.subcore_barrier()

    for r in range(RPS // RCH):
        row0 = sid * RPS + r * RCH
        pltpu.sync_copy(
            acc_sh.at[pl.ds(row0, RCH)],
            out_hbm.at[(pl.ds(row0, RCH), pl.ds(cid * DH, DH))])


def _make_spmm(cb):
    return functools.partial(
        pl.kernel,
        out_type=jax.ShapeDtypeStruct((N, D // 2), jnp.float32),
        mesh=_sc_mesh,
        scratch_types=[
            pltpu.VMEM((NB4, B4), jnp.int32),
            pltpu.VMEM((NB4, B4), jnp.int32),
            pltpu.VMEM((NB4, B4), jnp.float32),
            pltpu.VMEM((NB4, B4), jnp.float32),
            pltpu.VMEM((N,), jnp.float32),
            pltpu.VMEM((N,), jnp.float32),
            pltpu.VMEM((B4,), jnp.int32),
            pltpu.VMEM((B4, DH), jnp.float32),
            pltpu.VMEM((RCH, DH), jnp.float32),
            pltpu.VMEM_SHARED((N, DH), jnp.float32),
            pltpu.SemaphoreType.DMA,
        ],
        compiler_params=pltpu.CompilerParams(
            use_tc_tiling_on_sc=False, needs_layout_passes=False),
    )(functools.partial(_spmm_body, cb))


_spmm_kernels = [_make_spmm(0), _make_spmm(1)]


def kernel(emb_user, emb_item, eigs, lambda0s, path_embs, indices, path_type):
    all_emb = jnp.concatenate([emb_user, emb_item], axis=0)
    embs = [all_emb]
    for l in range(N_LAYERS):
        i0 = indices[l, 0]
        i1 = indices[l, 1]
        pt = path_type[l]
        i0_2 = i0.reshape(NW, NB2, B2)
        i1_2 = i1.reshape(NW, NB2, B2)
        pt_2 = pt.reshape(NW, NB2, B2)
        eigs_s = eigs * jnp.exp(lambda0s[l])
        pemb = jnp.zeros((L,), jnp.float32).at[:N_PATHS].set(
            path_embs[l].reshape(-1))
        aug0, aug1, y = _ln_aug(all_emb, eigs_s, eigs)
        e0, e1, p0, p1 = _edge_kernel(aug0, aug1, i0_2, i1_2, pt_2, pemb)
        inv = _combine(p0, p1)
        y4 = y.reshape(N * NQ, DH)
        i0_4 = i0.reshape(NS, NB4, B4)
        i1_4 = i1.reshape(NS, NB4, B4)
        e0_4 = e0.reshape(NS, NB4, B4)
        e1_4 = e1.reshape(NS, NB4, B4)
        outs = [k(y4, i0_4, i1_4, e0_4, e1_4, inv) for k in _spmm_kernels]
        all_emb = jnp.concatenate(outs, axis=1)
        embs.append(all_emb)
    return _mean3(embs[0], embs[1], embs[2])


# speedup vs baseline: 6.6224x; 1.1773x over previous
import functools

import jax
import jax.numpy as jnp
from jax import lax
from jax.experimental import pallas as pl
from jax.experimental.pallas import tpu as pltpu
from jax.experimental.pallas import tpu_sc as plsc

N_USERS = 5000
N_ITEMS = 5000
N = N_USERS + N_ITEMS
E = 160000
D = 256
EIGS_DIM = 16
N_LAYERS = 2
N_PATHS = 6
SQRT_DIM = 0.0625
EXP_OFF = 16.0

NC = 2
NS = 16
L = 16
NW = NC * NS


_LN_ROWS = 1000


DA = D + EIGS_DIM


def _ln_body(x_ref, eh_ref, a_ref, y_ref):
    x = x_ref[...]
    mu = jnp.mean(x, axis=-1, keepdims=True)
    var = jnp.mean((x - mu) ** 2, axis=-1, keepdims=True)
    y = (x - mu) * lax.rsqrt(var + 1e-5)
    y_ref[...] = y
    a_ref[:, :D] = y * 0.25
    a_ref[:, D:] = eh_ref[...]


def _ln_aug(x, eigs_h):
    return pl.pallas_call(
        _ln_body,
        out_shape=(
            jax.ShapeDtypeStruct((N, DA), jnp.float32),
            jax.ShapeDtypeStruct((N, D), jnp.float32),
        ),
        grid=(N // _LN_ROWS,),
        in_specs=[
            pl.BlockSpec((_LN_ROWS, D), lambda i: (i, 0)),
            pl.BlockSpec((_LN_ROWS, EIGS_DIM), lambda i: (i, 0)),
        ],
        out_specs=(
            pl.BlockSpec((_LN_ROWS, DA), lambda i: (i, 0)),
            pl.BlockSpec((_LN_ROWS, D), lambda i: (i, 0)),
        ),
    )(x, eigs_h)


def _combine_body(p0_ref, p1_ref, o_ref):
    s0 = jnp.sum(p0_ref[...], axis=0)
    s1 = jnp.sum(p1_ref[...], axis=0)
    o_ref[0, :] = 1.0 / (s0 + 1e-12)
    o_ref[1, :] = 1.0 / (s1 + 1e-12)


def _combine(p0, p1):
    return pl.pallas_call(
        _combine_body,
        out_shape=jax.ShapeDtypeStruct((2, N), jnp.float32),
    )(p0, p1)


def _mean_body(a_ref, b_ref, c_ref, o_ref):
    o_ref[...] = (a_ref[...] + b_ref[...] + c_ref[...]) * (1.0 / 3.0)


def _mean3(a, b, c):
    return pl.pallas_call(
        _mean_body,
        out_shape=jax.ShapeDtypeStruct((N, D), jnp.float32),
        grid=(N // _LN_ROWS,),
        in_specs=[pl.BlockSpec((_LN_ROWS, D), lambda i: (i, 0))] * 3,
        out_specs=pl.BlockSpec((_LN_ROWS, D), lambda i: (i, 0)),
    )(a, b, c)


EC2 = E // NW
B2 = 40
NB2 = EC2 // B2
B2P = 48

_sc_mesh = plsc.VectorSubcoreMesh(
    core_axis_name="c", subcore_axis_name="s", num_cores=NC, num_subcores=NS)


NBUF2 = 3


def _edge_body(aug_hbm, idx2_hbm, i0_hbm, pt_hbm, pemb_hbm,
               e0_hbm, e1_hbm, p0_hbm, p1_hbm,
               idx2_v, i0_v, pt_v, rows0, rows1, rows2,
               pemb_v, e0buf, e1buf, sums0, sums1, s0sem, s1sem, s2sem):
    cid = lax.axis_index("c")
    sid = lax.axis_index("s")
    wid = sid * NC + cid
    bufs = [(rows0, s0sem), (rows1, s1sem), (rows2, s2sem)]

    pltpu.sync_copy(idx2_hbm.at[wid], idx2_v)
    pltpu.sync_copy(i0_hbm.at[wid], i0_v)
    pltpu.sync_copy(pt_hbm.at[wid], pt_v)
    pltpu.sync_copy(pemb_hbm, pemb_v)

    zero = jnp.zeros((L,), jnp.float32)

    def _zero(i, _):
        sums0[pl.ds(i * L, L)] = zero
        sums1[pl.ds(i * L, L)] = zero
        return 0

    lax.fori_loop(0, N // L, _zero, 0)

    lane = jax.lax.iota(jnp.int32, L)
    groups = [(0, None), (L, None), (B2 - L, lane >= (3 * L - B2))]

    def _issue(j, b):
        pltpu.async_copy(aug_hbm.at[idx2_v.at[j]], bufs[b][0], bufs[b][1])

    def _wait(j, b):
        pltpu.make_async_copy(
            aug_hbm.at[idx2_v.at[j]], bufs[b][0], bufs[b][1]).wait()

    def _compute(j, b):
        r = bufs[b][0]
        for start, mk in groups:
            ev = lane + start

            def _dot(k, acc):
                kv = jnp.full((L,), 0, jnp.int32) + k
                g0v = plsc.load_gather(r, [ev, kv])
                g1v = plsc.load_gather(r, [ev + B2, kv])
                return acc + g0v * g1v

            s0v = lax.fori_loop(0, DA, _dot, jnp.zeros((L,), jnp.float32),
                                unroll=8)
            e0v = jnp.exp(s0v - EXP_OFF)
            sl_chunk = pl.ds(start, L)
            ptv = pt_v[j, sl_chunk]
            zv = plsc.load_gather(pemb_v, [ptv])
            e1v = jnp.exp(zv)
            if mk is None:
                e0buf[sl_chunk] = e0v
                e1buf[sl_chunk] = e1v
            else:
                e0buf[sl_chunk] = jnp.where(mk, e0v, e0buf[sl_chunk])
                e1buf[sl_chunk] = jnp.where(mk, e1v, e1buf[sl_chunk])
            i0v = i0_v[j, sl_chunk]
            plsc.addupdate_scatter(sums0, [i0v], e0v, mask=mk)
            plsc.addupdate_scatter(sums1, [i0v], e1v, mask=mk)

        pltpu.sync_copy(e0buf, e0_hbm.at[wid, j])
        pltpu.sync_copy(e1buf, e1_hbm.at[wid, j])

    for b in range(NBUF2 - 1):
        _issue(b, b)

    def _round(p, _):
        j0 = p * NBUF2
        for b in range(NBUF2):
            j = j0 + b

            @pl.when(j + NBUF2 - 1 < NB2)
            def _():
                _issue(j + NBUF2 - 1, (b + NBUF2 - 1) % NBUF2)

            @pl.when(j < NB2)
            def _():
                _wait(j, b)
                _compute(j, b)
        return 0

    lax.fori_loop(0, -(-NB2 // NBUF2), _round, 0)

    pltpu.sync_copy(sums0, p0_hbm.at[wid])
    pltpu.sync_copy(sums1, p1_hbm.at[wid])


_edge_kernel = functools.partial(
    pl.kernel,
    out_type=(
        jax.ShapeDtypeStruct((NW, NB2, B2), jnp.float32),
        jax.ShapeDtypeStruct((NW, NB2, B2), jnp.float32),
        jax.ShapeDtypeStruct((NW, N), jnp.float32),
        jax.ShapeDtypeStruct((NW, N), jnp.float32),
    ),
    mesh=_sc_mesh,
    scratch_types=[
        pltpu.VMEM((NB2, 2 * B2), jnp.int32),
        pltpu.VMEM((NB2, B2), jnp.int32),
        pltpu.VMEM((NB2, B2), jnp.int32),
        pltpu.VMEM((2 * B2, DA), jnp.float32),
        pltpu.VMEM((2 * B2, DA), jnp.float32),
        pltpu.VMEM((2 * B2, DA), jnp.float32),
        pltpu.VMEM((L,), jnp.float32),
        pltpu.VMEM((B2,), jnp.float32),
        pltpu.VMEM((B2,), jnp.float32),
        pltpu.VMEM((N,), jnp.float32),
        pltpu.VMEM((N,), jnp.float32),
        pltpu.SemaphoreType.DMA,
        pltpu.SemaphoreType.DMA,
        pltpu.SemaphoreType.DMA,
    ],
    compiler_params=pltpu.CompilerParams(use_tc_tiling_on_sc=False, needs_layout_passes=False),
)(_edge_body)


EC4 = E // NS
B4 = 80
NB4 = EC4 // B4
NQ = 4
DH = D // NQ
RPS = N // NS
RCH = 125


def _spmm_body(cb, y4_hbm, i0_hbm, i1_hbm, e0_hbm, e1_hbm, inv_hbm,
               out_hbm,
               i0_v, i1_v, e0_v, e1_v, inv0_v, inv1_v,
               ridxA, ridxB, rowsA, rowsB, zbuf, acc_sh, semA, semB):
    cid = lax.axis_index("c")
    sid = lax.axis_index("s")
    qid = cb * NC + cid
    bufs = [(ridxA, rowsA, semA), (ridxB, rowsB, semB)]

    zero = jnp.zeros((L,), jnp.float32)

    def _zero(i, _):
        zbuf[i // (DH // L), pl.ds((i % (DH // L)) * L, L)] = zero
        return 0

    lax.fori_loop(0, (RCH * DH) // L, _zero, 0)
    for r in range(RPS // RCH):
        pltpu.sync_copy(zbuf, acc_sh.at[pl.ds(sid * RPS + r * RCH, RCH)])
    plsc.subcore_barrier()

    pltpu.sync_copy(i0_hbm.at[sid], i0_v)
    pltpu.sync_copy(i1_hbm.at[sid], i1_v)
    pltpu.sync_copy(e0_hbm.at[sid], e0_v)
    pltpu.sync_copy(e1_hbm.at[sid], e1_v)
    pltpu.sync_copy(inv_hbm.at[0], inv0_v)
    pltpu.sync_copy(inv_hbm.at[1], inv1_v)

    def _issue(j, b):
        ridx, rows, sem = bufs[b]
        for t in range(B4 // L):
            i1v = i1_v[j, pl.ds(t * L, L)]
            ridx[pl.ds(t * L, L)] = i1v * NQ + qid
        pltpu.async_copy(y4_hbm.at[ridx], rows, sem)

    def _compute(j, b):
        ridx, rows, sem = bufs[b]
        pltpu.make_async_copy(y4_hbm.at[ridx], rows, sem).wait()
        for t in range(B4 // L):
            sl_chunk = pl.ds(t * L, L)
            i0v = i0_v[j, sl_chunk]
            a0 = plsc.load_gather(inv0_v, [i0v])
            a1 = plsc.load_gather(inv1_v, [i0v])
            sv = 0.5 * (e0_v[j, sl_chunk] * a0 + e1_v[j, sl_chunk] * a1)
            for e in range(L):
                se = sv[e]
                re = t * L + e
                for k in range(DH // L):
                    rows[re, pl.ds(k * L, L)] = rows[re, pl.ds(k * L, L)] * se
        pltpu.sync_copy(rows, acc_sh.at[i0_v.at[j]], add=True)

    _issue(0, 0)

    def _pair(p, _):
        j = 2 * p

        @pl.when(j + 1 < NB4)
        def _():
            _issue(j + 1, 1)

        _compute(j, 0)

        @pl.when(j + 2 < NB4)
        def _():
            _issue(j + 2, 0)

        @pl.when(j + 1 < NB4)
        def _():
            _compute(j + 1, 1)

        return 0

    lax.fori_loop(0, (NB4 + 1) // 2, _pair, 0)
    plsc.subcore_barrier()

    for r in range(RPS // RCH):
        row0 = sid * RPS + r * RCH
        pltpu.sync_copy(
            acc_sh.at[pl.ds(row0, RCH)],
            out_hbm.at[(pl.ds(row0, RCH), pl.ds(cid * DH, DH))])


def _make_spmm(cb):
    return functools.partial(
        pl.kernel,
        out_type=jax.ShapeDtypeStruct((N, D // 2), jnp.float32),
        mesh=_sc_mesh,
        scratch_types=[
            pltpu.VMEM((NB4, B4), jnp.int32),
            pltpu.VMEM((NB4, B4), jnp.int32),
            pltpu.VMEM((NB4, B4), jnp.float32),
            pltpu.VMEM((NB4, B4), jnp.float32),
            pltpu.VMEM((N,), jnp.float32),
            pltpu.VMEM((N,), jnp.float32),
            pltpu.VMEM((B4,), jnp.int32),
            pltpu.VMEM((B4,), jnp.int32),
            pltpu.VMEM((B4, DH), jnp.float32),
            pltpu.VMEM((B4, DH), jnp.float32),
            pltpu.VMEM((RCH, DH), jnp.float32),
            pltpu.VMEM_SHARED((N, DH), jnp.float32),
            pltpu.SemaphoreType.DMA,
            pltpu.SemaphoreType.DMA,
        ],
        compiler_params=pltpu.CompilerParams(
            use_tc_tiling_on_sc=False, needs_layout_passes=False),
    )(functools.partial(_spmm_body, cb))


_spmm_kernels = [_make_spmm(0), _make_spmm(1)]


def kernel(emb_user, emb_item, eigs, lambda0s, path_embs, indices, path_type):
    all_emb = jnp.concatenate([emb_user, emb_item], axis=0)
    embs = [all_emb]
    for l in range(N_LAYERS):
        i0 = indices[l, 0]
        i1 = indices[l, 1]
        pt = path_type[l]
        i0_2 = i0.reshape(NW, NB2, B2)
        i1_2 = i1.reshape(NW, NB2, B2)
        idx2 = jnp.concatenate([i0_2, i1_2], axis=2)
        pt_2 = pt.reshape(NW, NB2, B2)
        eigs_h = eigs * jnp.exp(0.5 * lambda0s[l])
        pemb = jnp.zeros((L,), jnp.float32).at[:N_PATHS].set(
            path_embs[l].reshape(-1))
        aug, y = _ln_aug(all_emb, eigs_h)
        e0, e1, p0, p1 = _edge_kernel(aug, idx2, i0_2, pt_2, pemb)
        inv = _combine(p0, p1)
        y4 = y.reshape(N * NQ, DH)
        i0_4 = i0.reshape(NS, NB4, B4)
        i1_4 = i1.reshape(NS, NB4, B4)
        e0_4 = e0.reshape(NS, NB4, B4)
        e1_4 = e1.reshape(NS, NB4, B4)
        outs = [k(y4, i0_4, i1_4, e0_4, e1_4, inv) for k in _spmm_kernels]
        all_emb = jnp.concatenate(outs, axis=1)
        embs.append(all_emb)
    return _mean3(embs[0], embs[1], embs[2])


# speedup vs baseline: 9.8148x; 1.4821x over previous
import functools

import jax
import jax.numpy as jnp
from jax import lax
from jax.experimental import pallas as pl
from jax.experimental.pallas import tpu as pltpu
from jax.experimental.pallas import tpu_sc as plsc

N_USERS = 5000
N_ITEMS = 5000
N = N_USERS + N_ITEMS
E = 160000
D = 256
EIGS_DIM = 16
N_LAYERS = 2
N_PATHS = 6
SQRT_DIM = 0.0625
EXP_OFF = 16.0

NC = 2
NS = 16
L = 16
NW = NC * NS


_LN_ROWS = 1000


DA = D + EIGS_DIM
DAP = DA // 2


def _ln_body(x_ref, eh_ref, a_ref, y_ref):
    x = x_ref[...]
    mu = jnp.mean(x, axis=-1, keepdims=True)
    var = jnp.mean((x - mu) ** 2, axis=-1, keepdims=True)
    y = (x - mu) * lax.rsqrt(var + 1e-5)
    y_ref[...] = y
    a_ref[:, :D] = y * 0.25
    a_ref[:, D:] = eh_ref[...]


def _ln_aug(x, eigs_h):
    return pl.pallas_call(
        _ln_body,
        out_shape=(
            jax.ShapeDtypeStruct((N, DA), jnp.float32),
            jax.ShapeDtypeStruct((N, D), jnp.float32),
        ),
        grid=(N // _LN_ROWS,),
        in_specs=[
            pl.BlockSpec((_LN_ROWS, D), lambda i: (i, 0)),
            pl.BlockSpec((_LN_ROWS, EIGS_DIM), lambda i: (i, 0)),
        ],
        out_specs=(
            pl.BlockSpec((_LN_ROWS, DA), lambda i: (i, 0)),
            pl.BlockSpec((_LN_ROWS, D), lambda i: (i, 0)),
        ),
    )(x, eigs_h)


def _combine_body(p0_ref, p1_ref, o_ref):
    s0 = jnp.sum(p0_ref[...], axis=0)
    s1 = jnp.sum(p1_ref[...], axis=0)
    o_ref[0, :] = 1.0 / (s0 + 1e-12)
    o_ref[1, :] = 1.0 / (s1 + 1e-12)


def _combine(p0, p1):
    return pl.pallas_call(
        _combine_body,
        out_shape=jax.ShapeDtypeStruct((2, N), jnp.float32),
    )(p0, p1)


def _mean_body(a_ref, b_ref, c_ref, o_ref):
    o_ref[...] = (a_ref[...] + b_ref[...] + c_ref[...]) * (1.0 / 3.0)


def _mean3(a, b, c):
    return pl.pallas_call(
        _mean_body,
        out_shape=jax.ShapeDtypeStruct((N, D), jnp.float32),
        grid=(N // _LN_ROWS,),
        in_specs=[pl.BlockSpec((_LN_ROWS, D), lambda i: (i, 0))] * 3,
        out_specs=pl.BlockSpec((_LN_ROWS, D), lambda i: (i, 0)),
    )(a, b, c)


EC2 = E // NW
B2 = 40
NB2 = EC2 // B2
B2P = 48

_sc_mesh = plsc.VectorSubcoreMesh(
    core_axis_name="c", subcore_axis_name="s", num_cores=NC, num_subcores=NS)


NBUF2 = 3


def _edge_body(aug_hbm, idx2_hbm, i0_hbm, pt_hbm, pemb_hbm,
               e0_hbm, e1_hbm, p0_hbm, p1_hbm,
               idx2_v, i0_v, pt_v, rows0, rows1, rows2,
               pemb_v, e0buf, e1buf, sums0, sums1, s0sem, s1sem, s2sem):
    cid = lax.axis_index("c")
    sid = lax.axis_index("s")
    wid = sid * NC + cid
    bufs = [(rows0, s0sem), (rows1, s1sem), (rows2, s2sem)]

    pltpu.sync_copy(idx2_hbm.at[wid], idx2_v)
    pltpu.sync_copy(i0_hbm.at[wid], i0_v)
    pltpu.sync_copy(pt_hbm.at[wid], pt_v)
    pltpu.sync_copy(pemb_hbm, pemb_v)

    zero = jnp.zeros((L,), jnp.float32)

    def _zero(i, _):
        sums0[pl.ds(i * L, L)] = zero
        sums1[pl.ds(i * L, L)] = zero
        return 0

    lax.fori_loop(0, N // L, _zero, 0)

    lane = jax.lax.iota(jnp.int32, L)
    groups = [(0, None), (L, None), (B2 - L, lane >= (3 * L - B2))]

    def _issue(j, b):
        pltpu.async_copy(aug_hbm.at[idx2_v.at[j]], bufs[b][0], bufs[b][1])

    def _wait(j, b):
        pltpu.make_async_copy(
            aug_hbm.at[idx2_v.at[j]], bufs[b][0], bufs[b][1]).wait()

    def _compute(j, b):
        r = bufs[b][0]
        for start, mk in groups:
            ev = lane + start

            def _dot(k, accs):
                acc_a, acc_b = accs
                kv = jnp.full((L,), 0, jnp.int32) + k
                g0v = plsc.bitcast(plsc.load_gather(r, [ev, kv]),
                                   jnp.bfloat16)
                g1v = plsc.bitcast(plsc.load_gather(r, [ev + B2, kv]),
                                   jnp.bfloat16)
                a0, c0 = plsc.unpack(g0v, format=plsc.PackFormat.INTERLEAVED)
                a1, c1 = plsc.unpack(g1v, format=plsc.PackFormat.INTERLEAVED)
                return (acc_a + a0 * a1, acc_b + c0 * c1)

            za, zb = lax.fori_loop(
                0, DAP, _dot,
                (jnp.zeros((L,), jnp.float32), jnp.zeros((L,), jnp.float32)),
                unroll=8)
            s0v = za + zb
            e0v = jnp.exp(s0v - EXP_OFF)
            sl_chunk = pl.ds(start, L)
            ptv = pt_v[j, sl_chunk]
            zv = plsc.load_gather(pemb_v, [ptv])
            e1v = jnp.exp(zv)
            if mk is None:
                e0buf[sl_chunk] = e0v
                e1buf[sl_chunk] = e1v
            else:
                e0buf[sl_chunk] = jnp.where(mk, e0v, e0buf[sl_chunk])
                e1buf[sl_chunk] = jnp.where(mk, e1v, e1buf[sl_chunk])
            i0v = i0_v[j, sl_chunk]
            plsc.addupdate_scatter(sums0, [i0v], e0v, mask=mk)
            plsc.addupdate_scatter(sums1, [i0v], e1v, mask=mk)

        pltpu.sync_copy(e0buf, e0_hbm.at[wid, j])
        pltpu.sync_copy(e1buf, e1_hbm.at[wid, j])

    for b in range(NBUF2 - 1):
        _issue(b, b)

    def _round(p, _):
        j0 = p * NBUF2
        for b in range(NBUF2):
            j = j0 + b

            @pl.when(j + NBUF2 - 1 < NB2)
            def _():
                _issue(j + NBUF2 - 1, (b + NBUF2 - 1) % NBUF2)

            @pl.when(j < NB2)
            def _():
                _wait(j, b)
                _compute(j, b)
        return 0

    lax.fori_loop(0, -(-NB2 // NBUF2), _round, 0)

    pltpu.sync_copy(sums0, p0_hbm.at[wid])
    pltpu.sync_copy(sums1, p1_hbm.at[wid])


_edge_kernel = functools.partial(
    pl.kernel,
    out_type=(
        jax.ShapeDtypeStruct((NW, NB2, B2), jnp.float32),
        jax.ShapeDtypeStruct((NW, NB2, B2), jnp.float32),
        jax.ShapeDtypeStruct((NW, N), jnp.float32),
        jax.ShapeDtypeStruct((NW, N), jnp.float32),
    ),
    mesh=_sc_mesh,
    scratch_types=[
        pltpu.VMEM((NB2, 2 * B2), jnp.int32),
        pltpu.VMEM((NB2, B2), jnp.int32),
        pltpu.VMEM((NB2, B2), jnp.int32),
        pltpu.VMEM((2 * B2, DAP), jnp.int32),
        pltpu.VMEM((2 * B2, DAP), jnp.int32),
        pltpu.VMEM((2 * B2, DAP), jnp.int32),
        pltpu.VMEM((L,), jnp.float32),
        pltpu.VMEM((B2,), jnp.float32),
        pltpu.VMEM((B2,), jnp.float32),
        pltpu.VMEM((N,), jnp.float32),
        pltpu.VMEM((N,), jnp.float32),
        pltpu.SemaphoreType.DMA,
        pltpu.SemaphoreType.DMA,
        pltpu.SemaphoreType.DMA,
    ],
    compiler_params=pltpu.CompilerParams(use_tc_tiling_on_sc=False, needs_layout_passes=False),
)(_edge_body)


EC4 = E // NS
B4 = 80
NB4 = EC4 // B4
NQ = 4
DH = D // NQ
RPS = N // NS
RCH = 125


def _spmm_body(cb, y4_hbm, i0_hbm, i1_hbm, e0_hbm, e1_hbm, inv_hbm,
               out_hbm,
               i0_v, i1_v, e0_v, e1_v, inv0_v, inv1_v,
               ridxA, ridxB, rowsA, rowsB, zbuf, acc_sh, semA, semB):
    cid = lax.axis_index("c")
    sid = lax.axis_index("s")
    qid = cb * NC + cid
    bufs = [(ridxA, rowsA, semA), (ridxB, rowsB, semB)]

    zero = jnp.zeros((L,), jnp.float32)

    def _zero(i, _):
        zbuf[i // (DH // L), pl.ds((i % (DH // L)) * L, L)] = zero
        return 0

    lax.fori_loop(0, (RCH * DH) // L, _zero, 0)
    for r in range(RPS // RCH):
        pltpu.sync_copy(zbuf, acc_sh.at[pl.ds(sid * RPS + r * RCH, RCH)])
    plsc.subcore_barrier()

    pltpu.sync_copy(i0_hbm.at[sid], i0_v)
    pltpu.sync_copy(i1_hbm.at[sid], i1_v)
    pltpu.sync_copy(e0_hbm.at[sid], e0_v)
    pltpu.sync_copy(e1_hbm.at[sid], e1_v)
    pltpu.sync_copy(inv_hbm.at[0], inv0_v)
    pltpu.sync_copy(inv_hbm.at[1], inv1_v)

    def _issue(j, b):
        ridx, rows, sem = bufs[b]
        for t in range(B4 // L):
            i1v = i1_v[j, pl.ds(t * L, L)]
            ridx[pl.ds(t * L, L)] = i1v * NQ + qid
        pltpu.async_copy(y4_hbm.at[ridx], rows, sem)

    def _compute(j, b):
        ridx, rows, sem = bufs[b]
        pltpu.make_async_copy(y4_hbm.at[ridx], rows, sem).wait()
        for t in range(B4 // L):
            sl_chunk = pl.ds(t * L, L)
            i0v = i0_v[j, sl_chunk]
            a0 = plsc.load_gather(inv0_v, [i0v])
            a1 = plsc.load_gather(inv1_v, [i0v])
            sv = 0.5 * (e0_v[j, sl_chunk] * a0 + e1_v[j, sl_chunk] * a1)
            for e in range(L):
                se = sv[e]
                re = t * L + e
                for k in range(DH // L):
                    rows[re, pl.ds(k * L, L)] = rows[re, pl.ds(k * L, L)] * se
        pltpu.sync_copy(rows, acc_sh.at[i0_v.at[j]], add=True)

    _issue(0, 0)

    def _pair(p, _):
        j = 2 * p

        @pl.when(j + 1 < NB4)
        def _():
            _issue(j + 1, 1)

        _compute(j, 0)

        @pl.when(j + 2 < NB4)
        def _():
            _issue(j + 2, 0)

        @pl.when(j + 1 < NB4)
        def _():
            _compute(j + 1, 1)

        return 0

    lax.fori_loop(0, (NB4 + 1) // 2, _pair, 0)
    plsc.subcore_barrier()

    for r in range(RPS // RCH):
        row0 = sid * RPS + r * RCH
        pltpu.sync_copy(
            acc_sh.at[pl.ds(row0, RCH)],
            out_hbm.at[(pl.ds(row0, RCH), pl.ds(cid * DH, DH))])


def _make_spmm(cb):
    return functools.partial(
        pl.kernel,
        out_type=jax.ShapeDtypeStruct((N, D // 2), jnp.float32),
        mesh=_sc_mesh,
        scratch_types=[
            pltpu.VMEM((NB4, B4), jnp.int32),
            pltpu.VMEM((NB4, B4), jnp.int32),
            pltpu.VMEM((NB4, B4), jnp.float32),
            pltpu.VMEM((NB4, B4), jnp.float32),
            pltpu.VMEM((N,), jnp.float32),
            pltpu.VMEM((N,), jnp.float32),
            pltpu.VMEM((B4,), jnp.int32),
            pltpu.VMEM((B4,), jnp.int32),
            pltpu.VMEM((B4, DH), jnp.float32),
            pltpu.VMEM((B4, DH), jnp.float32),
            pltpu.VMEM((RCH, DH), jnp.float32),
            pltpu.VMEM_SHARED((N, DH), jnp.float32),
            pltpu.SemaphoreType.DMA,
            pltpu.SemaphoreType.DMA,
        ],
        compiler_params=pltpu.CompilerParams(
            use_tc_tiling_on_sc=False, needs_layout_passes=False),
    )(functools.partial(_spmm_body, cb))


_spmm_kernels = [_make_spmm(0), _make_spmm(1)]


def kernel(emb_user, emb_item, eigs, lambda0s, path_embs, indices, path_type):
    all_emb = jnp.concatenate([emb_user, emb_item], axis=0)
    embs = [all_emb]
    for l in range(N_LAYERS):
        i0 = indices[l, 0]
        i1 = indices[l, 1]
        pt = path_type[l]
        i0_2 = i0.reshape(NW, NB2, B2)
        i1_2 = i1.reshape(NW, NB2, B2)
        idx2 = jnp.concatenate([i0_2, i1_2], axis=2)
        pt_2 = pt.reshape(NW, NB2, B2)
        eigs_h = eigs * jnp.exp(0.5 * lambda0s[l])
        pemb = jnp.zeros((L,), jnp.float32).at[:N_PATHS].set(
            path_embs[l].reshape(-1))
        aug, y = _ln_aug(all_emb, eigs_h)
        augp = lax.bitcast_convert_type(
            aug.astype(jnp.bfloat16).reshape(N, DAP, 2), jnp.int32)
        e0, e1, p0, p1 = _edge_kernel(augp, idx2, i0_2, pt_2, pemb)
        inv = _combine(p0, p1)
        y4 = y.reshape(N * NQ, DH)
        i0_4 = i0.reshape(NS, NB4, B4)
        i1_4 = i1.reshape(NS, NB4, B4)
        e0_4 = e0.reshape(NS, NB4, B4)
        e1_4 = e1.reshape(NS, NB4, B4)
        outs = [k(y4, i0_4, i1_4, e0_4, e1_4, inv) for k in _spmm_kernels]
        all_emb = jnp.concatenate(outs, axis=1)
        embs.append(all_emb)
    return _mean3(embs[0], embs[1], embs[2])
